# Initial kernel scaffold; baseline (speedup 1.0000x reference)
import jax
import jax.numpy as jnp
from jax.experimental import pallas as pl


def kernel(edge_indices, edge_features, var_features, con_features, params):
    raise NotImplementedError("write your pallas kernel here")


# speedup vs baseline: 34.1441x; 34.1441x over previous
import functools

import jax
import jax.numpy as jnp
import numpy as np
from jax import lax
from jax.experimental import pallas as pl
from jax.experimental.pallas import tpu as pltpu
from jax.experimental.pallas import tpu_sc as plsc

H, DM, DK, DV, DFF = 8, 128, 16, 16, 512
NV, NE = 10000, 320000
NB = 1000
EB = 2000
GW = 128
SW = 80
NPAD = 10240
MR = 144
NSC, NSUB = 2, 16
EPW = NE // (NSC * NSUB)

_f32 = jnp.float32


def _ln(y, g, b):
    m = jnp.mean(y, axis=-1, keepdims=True)
    v = jnp.mean((y - m) ** 2, axis=-1, keepdims=True)
    return (y - m) * jax.lax.rsqrt(v + 1e-5) * g + b


def _sa1_body(x_ref, wq_ref, wk_ref, wv_ref, q_ref, kv_ref, ks_ref,
              acc_kv, acc_ks):
    i = pl.program_id(0)

    @pl.when(i == 0)
    def _():
        acc_kv[...] = jnp.zeros_like(acc_kv)
        acc_ks[...] = jnp.zeros_like(acc_ks)

    x = x_ref[...]
    q = jax.nn.sigmoid(jnp.dot(x, wq_ref[...], preferred_element_type=_f32))
    k = jax.nn.sigmoid(jnp.dot(x, wk_ref[...], preferred_element_type=_f32))
    v = jnp.dot(x, wv_ref[...], preferred_element_type=_f32)
    q_ref[...] = q
    acc_kv[...] += lax.dot_general(k, v, (((0,), (0,)), ((), ())),
                                   preferred_element_type=_f32)
    acc_ks[...] += jnp.broadcast_to(jnp.sum(k, axis=0, keepdims=True), (8, DM))

    @pl.when(i == NV // NB - 1)
    def _():
        kv_ref[...] = acc_kv[...]
        ks_ref[...] = acc_ks[...]


def _sa2_body(q_ref, x_ref, kv_ref, ks_ref, fc_ref, lnp_ref, bd_ref, o_ref):
    q = q_ref[...]
    kvm = kv_ref[...] * bd_ref[...]
    num = jnp.dot(q, kvm, preferred_element_type=_f32)
    den = jnp.dot(q * ks_ref[0:1, :], bd_ref[...], preferred_element_type=_f32)
    out = num / (den + 1e-8)
    y = jnp.dot(out, fc_ref[...], preferred_element_type=_f32) + x_ref[...]
    o_ref[...] = _ln(y, lnp_ref[0:1, :], lnp_ref[1:2, :])


def _self_attn(x, p, bd):
    grid = (NV // NB,)
    row = lambda i: (i, 0)
    full = lambda i: (0, 0)
    q, kv, ks = pl.pallas_call(
        _sa1_body,
        grid=grid,
        in_specs=[pl.BlockSpec((NB, DM), row)] + [pl.BlockSpec((DM, DM), full)] * 3,
        out_specs=[pl.BlockSpec((NB, DM), row), pl.BlockSpec((DM, DM), full),
                   pl.BlockSpec((8, DM), full)],
        out_shape=[jax.ShapeDtypeStruct((NV, DM), _f32),
                   jax.ShapeDtypeStruct((DM, DM), _f32),
                   jax.ShapeDtypeStruct((8, DM), _f32)],
        scratch_shapes=[pltpu.VMEM((DM, DM), _f32), pltpu.VMEM((8, DM), _f32)],
    )(x, p['WQ'], p['WK'], p['WV'])
    lnp = jnp.stack([p['ln_g'], p['ln_b']])
    out = pl.pallas_call(
        _sa2_body,
        grid=grid,
        in_specs=[pl.BlockSpec((NB, DM), row), pl.BlockSpec((NB, DM), row),
                  pl.BlockSpec((DM, DM), full), pl.BlockSpec((8, DM), full),
                  pl.BlockSpec((DM, DM), full), pl.BlockSpec((2, DM), full),
                  pl.BlockSpec((DM, DM), full)],
        out_specs=pl.BlockSpec((NB, DM), row),
        out_shape=jax.ShapeDtypeStruct((NV, DM), _f32),
    )(q, x, kv, ks, p['fc'], lnp, bd)
    return out


def _projkvq_body(x_ref, wk_ref, wv_ref, wq_ref, kv_ref, q_ref):
    x = x_ref[...]
    kv_ref[:, 0:DM] = jnp.dot(x, wk_ref[...], preferred_element_type=_f32)
    kv_ref[:, DM:2 * DM] = jnp.dot(x, wv_ref[...], preferred_element_type=_f32)
    q_ref[...] = jnp.dot(x, wq_ref[...], preferred_element_type=_f32)


def _proj_kv_q(x, wk, wv, wq):
    row = lambda i: (i, 0)
    full = lambda i: (0, 0)
    return pl.pallas_call(
        _projkvq_body,
        grid=(NV // NB,),
        in_specs=[pl.BlockSpec((NB, DM), row)] + [pl.BlockSpec((DM, DM), full)] * 3,
        out_specs=[pl.BlockSpec((NB, 2 * DM), row), pl.BlockSpec((NB, DM), row)],
        out_shape=[jax.ShapeDtypeStruct((NV, 2 * DM), _f32),
                   jax.ShapeDtypeStruct((NV, DM), _f32)],
    )(x, wk, wv, wq)


def _proj2_body(x_ref, wa_ref, wb_ref, oa_ref, ob_ref):
    x = x_ref[...]
    oa_ref[...] = jnp.dot(x, wa_ref[...], preferred_element_type=_f32)
    ob_ref[...] = jnp.dot(x, wb_ref[...], preferred_element_type=_f32)


def _proj2(x, wa, wb):
    row = lambda i: (i, 0)
    full = lambda i: (0, 0)
    return pl.pallas_call(
        _proj2_body,
        grid=(NE // EB,),
        in_specs=[pl.BlockSpec((EB, DM), row)] + [pl.BlockSpec((DM, DM), full)] * 2,
        out_specs=[pl.BlockSpec((EB, DM), row)] * 2,
        out_shape=[jax.ShapeDtypeStruct((NE, DM), _f32)] * 2,
    )(x, wa, wb)


def _edge_body(q_ref, kv_ref, e_ref, bd_ref, o_ref):
    q = q_ref[...]
    k = kv_ref[:, 0:DM]
    v = kv_ref[:, DM:2 * DM]
    s = q * k * e_ref[...]
    srep = jnp.dot(s, bd_ref[...], preferred_element_type=_f32) * 0.25
    attn = jnp.exp(jnp.clip(srep, -5.0, 5.0))
    o_ref[0] = attn * v
    o_ref[1] = attn


def _edge_math(qs, kvt, e, bd):
    row = lambda i: (i, 0)
    full = lambda i: (0, 0)
    return pl.pallas_call(
        _edge_body,
        grid=(NE // EB,),
        in_specs=[pl.BlockSpec((EB, DM), row), pl.BlockSpec((EB, 2 * DM), row),
                  pl.BlockSpec((EB, DM), row), pl.BlockSpec((DM, DM), full)],
        out_specs=pl.BlockSpec((2, EB, DM), lambda i: (0, i, 0)),
        out_shape=jax.ShapeDtypeStruct((2, NE, DM), _f32),
    )(qs, kvt, e, bd)


def _sc_gather(qtab, kvtab, qidx, kvidx):
    mesh = plsc.VectorSubcoreMesh(core_axis_name="c", subcore_axis_name="s")

    @functools.partial(
        pl.kernel,
        out_type=[jax.ShapeDtypeStruct((NE, DM), _f32),
                  jax.ShapeDtypeStruct((NE, 2 * DM), _f32)],
        mesh=mesh,
    )
    def gk(q_hbm, kv_hbm, qi_hbm, ki_hbm, oq_hbm, okv_hbm):
        def body(qi_v, ki_v, oq_v, okv_v):
            pltpu.sync_copy(q_hbm.at[qi_v.at[0]], oq_v)
            pltpu.sync_copy(kv_hbm.at[ki_v.at[0]], okv_v)

        pltpu.emit_pipeline(
            body,
            grid=(NE // GW,),
            in_specs=[pl.BlockSpec((1, GW), lambda i: (0, i)),
                      pl.BlockSpec((1, GW), lambda i: (0, i))],
            out_specs=[pl.BlockSpec((GW, DM), lambda i: (i, 0)),
                       pl.BlockSpec((GW, 2 * DM), lambda i: (i, 0))],
            core_axis_name=("c", "s"),
            dimension_semantics=(pltpu.PARALLEL,),
        )(qi_hbm, ki_hbm, oq_hbm, okv_hbm)

    return gk(qtab, kvtab, qidx.reshape(1, NE), kvidx.reshape(1, NE))


def _sc_scatter(msg2, sidx):
    mesh = plsc.VectorSubcoreMesh(core_axis_name="c", subcore_axis_name="s")
    epw = NE // NSUB

    @functools.partial(
        pl.kernel,
        out_type=jax.ShapeDtypeStruct((2, NPAD, DM), _f32),
        mesh=mesh,
        scratch_types=[
            pltpu.VMEM((SW,), jnp.int32),
            pltpu.VMEM((SW, DM), _f32),
            pltpu.VMEM((128, DM), _f32),
            pltpu.VMEM_SHARED((NPAD, DM), _f32),
        ],
    )
    def sk(m_hbm, si_hbm, o_hbm, idx_v, row_v, z_v, acc_s):
        c = lax.axis_index("c")
        s = lax.axis_index("s")

        @pl.loop(0, 128)
        def _(r):
            for cc in range(DM // 16):
                z_v[r, pl.ds(cc * 16, 16)] = jnp.zeros((16,), _f32)

        @pl.loop(0, NPAD // NSUB // 128)
        def _(z):
            pltpu.sync_copy(z_v, acc_s.at[pl.ds(s * (NPAD // NSUB) + z * 128, 128)])

        plsc.subcore_barrier()

        base = s * epw

        @pl.loop(0, epw // SW)
        def _(j):
            b = base + j * SW
            pltpu.sync_copy(si_hbm.at[pl.ds(b, SW)], idx_v)
            pltpu.sync_copy(m_hbm.at[c, pl.ds(b, SW)], row_v)
            pltpu.sync_copy(row_v, acc_s.at[idx_v], add=True)

        plsc.subcore_barrier()
        r0 = s * (NPAD // NSUB)
        pltpu.sync_copy(acc_s.at[pl.ds(r0, NPAD // NSUB)],
                        o_hbm.at[c, pl.ds(r0, NPAD // NSUB)])

    return sk(msg2, sidx)


def _post_body(m_ref, a_ref, xq_ref, fc_ref, lnp_ref,
               fc1_ref, fc2_ref, ln2_ref, o_ref):
    out = m_ref[0] / (a_ref[0] + 1e-8)
    y = jnp.dot(out, fc_ref[...], preferred_element_type=_f32) + xq_ref[...]
    y = _ln(y, lnp_ref[0:1, :], lnp_ref[1:2, :])
    h = jnp.maximum(jnp.dot(y, fc1_ref[...], preferred_element_type=_f32), 0.0)
    z = jnp.dot(h, fc2_ref[...], preferred_element_type=_f32) + y
    o_ref[...] = _ln(z, ln2_ref[0:1, :], ln2_ref[1:2, :])


def _post_ffn(acc, xq, pca, pffn):
    row = lambda i: (i, 0)
    full = lambda i: (0, 0)
    lnp = jnp.stack([pca['ln_g'], pca['ln_b']])
    ln2 = jnp.stack([pffn['ln_g'], pffn['ln_b']])
    return pl.pallas_call(
        _post_body,
        grid=(NV // NB,),
        in_specs=[pl.BlockSpec((1, NB, DM), lambda i: (0, i, 0)),
                  pl.BlockSpec((1, NB, DM), lambda i: (1, i, 0)),
                  pl.BlockSpec((NB, DM), row),
                  pl.BlockSpec((DM, DM), full),
                  pl.BlockSpec((2, DM), full),
                  pl.BlockSpec((DM, DFF), full),
                  pl.BlockSpec((DFF, DM), full),
                  pl.BlockSpec((2, DM), full)],
        out_specs=pl.BlockSpec((NB, DM), row),
        out_shape=jax.ShapeDtypeStruct((NV, DM), _f32),
    )(acc, acc, xq, pca['fc'], lnp, pffn['fc1'], pffn['fc2'], ln2)


def kernel(edge_indices, edge_features, var_features, con_features, params):
    src = edge_indices[0, 0]
    tgt = edge_indices[0, 1]
    ef = edge_features[0]
    xv = var_features[0]
    xc = con_features[0]
    p = params

    hid = np.arange(DM) // DK
    bd = jnp.asarray((hid[:, None] == hid[None, :]).astype(np.float32))

    var1 = _self_attn(xv, p['sa_var'], bd)
    con1 = _self_attn(xc, p['sa_con'], bd)

    kv_v2c, q_c2v = _proj_kv_q(var1, p['ca_v2c']['WK'], p['ca_v2c']['WV'],
                               p['ca_c2v']['WQ'])
    kv_c2v, q_v2c = _proj_kv_q(con1, p['ca_c2v']['WK'], p['ca_c2v']['WV'],
                               p['ca_v2c']['WQ'])
    e_v2c, e_c2v = _proj2(ef, p['ca_v2c']['WE'], p['ca_c2v']['WE'])

    qs_v2c, kvt_v2c = _sc_gather(q_v2c, kv_v2c, src, tgt)
    msg_v2c = _edge_math(qs_v2c, kvt_v2c, e_v2c, bd)
    acc_v2c = _sc_scatter(msg_v2c, src)

    qs_c2v, kvt_c2v = _sc_gather(q_c2v, kv_c2v, tgt, src)
    msg_c2v = _edge_math(qs_c2v, kvt_c2v, e_c2v, bd)
    acc_c2v = _sc_scatter(msg_c2v, tgt)

    con_out = _post_ffn(acc_v2c, con1, p['ca_v2c'], p['ffn_con'])
    var_out = _post_ffn(acc_c2v, var1, p['ca_c2v'], p['ffn_var'])

    return (var_out[None], con_out[None])


# speedup vs baseline: 44.2890x; 1.2971x over previous
import functools

import jax
import jax.numpy as jnp
import numpy as np
from jax import lax
from jax.experimental import pallas as pl
from jax.experimental.pallas import tpu as pltpu
from jax.experimental.pallas import tpu_sc as plsc

H, DM, DK, DV, DFF = 8, 128, 16, 16, 512
NV, NE = 10000, 320000
NB = 1000
EB = 2000
GW = 128
SW = 128
NPAD = 10240
MR = 144
NSC, NSUB = 2, 16
EPW = NE // (NSC * NSUB)

_f32 = jnp.float32


def _ln(y, g, b):
    m = jnp.mean(y, axis=-1, keepdims=True)
    v = jnp.mean((y - m) ** 2, axis=-1, keepdims=True)
    return (y - m) * jax.lax.rsqrt(v + 1e-5) * g + b


def _sa1_body(x_ref, wq_ref, wk_ref, wv_ref, q_ref, kv_ref, ks_ref,
              acc_kv, acc_ks):
    i = pl.program_id(0)

    @pl.when(i == 0)
    def _():
        acc_kv[...] = jnp.zeros_like(acc_kv)
        acc_ks[...] = jnp.zeros_like(acc_ks)

    x = x_ref[...]
    q = jax.nn.sigmoid(jnp.dot(x, wq_ref[...], preferred_element_type=_f32))
    k = jax.nn.sigmoid(jnp.dot(x, wk_ref[...], preferred_element_type=_f32))
    v = jnp.dot(x, wv_ref[...], preferred_element_type=_f32)
    q_ref[...] = q
    acc_kv[...] += lax.dot_general(k, v, (((0,), (0,)), ((), ())),
                                   preferred_element_type=_f32)
    acc_ks[...] += jnp.broadcast_to(jnp.sum(k, axis=0, keepdims=True), (8, DM))

    @pl.when(i == NV // NB - 1)
    def _():
        kv_ref[...] = acc_kv[...]
        ks_ref[...] = acc_ks[...]


def _sa2_body(q_ref, x_ref, kv_ref, ks_ref, fc_ref, lnp_ref, bd_ref, o_ref):
    q = q_ref[...]
    kvm = kv_ref[...] * bd_ref[...]
    num = jnp.dot(q, kvm, preferred_element_type=_f32)
    den = jnp.dot(q * ks_ref[0:1, :], bd_ref[...], preferred_element_type=_f32)
    out = num / (den + 1e-8)
    y = jnp.dot(out, fc_ref[...], preferred_element_type=_f32) + x_ref[...]
    o_ref[...] = _ln(y, lnp_ref[0:1, :], lnp_ref[1:2, :])


def _self_attn(x, p, bd):
    grid = (NV // NB,)
    row = lambda i: (i, 0)
    full = lambda i: (0, 0)
    q, kv, ks = pl.pallas_call(
        _sa1_body,
        grid=grid,
        in_specs=[pl.BlockSpec((NB, DM), row)] + [pl.BlockSpec((DM, DM), full)] * 3,
        out_specs=[pl.BlockSpec((NB, DM), row), pl.BlockSpec((DM, DM), full),
                   pl.BlockSpec((8, DM), full)],
        out_shape=[jax.ShapeDtypeStruct((NV, DM), _f32),
                   jax.ShapeDtypeStruct((DM, DM), _f32),
                   jax.ShapeDtypeStruct((8, DM), _f32)],
        scratch_shapes=[pltpu.VMEM((DM, DM), _f32), pltpu.VMEM((8, DM), _f32)],
    )(x, p['WQ'], p['WK'], p['WV'])
    lnp = jnp.stack([p['ln_g'], p['ln_b']])
    out = pl.pallas_call(
        _sa2_body,
        grid=grid,
        in_specs=[pl.BlockSpec((NB, DM), row), pl.BlockSpec((NB, DM), row),
                  pl.BlockSpec((DM, DM), full), pl.BlockSpec((8, DM), full),
                  pl.BlockSpec((DM, DM), full), pl.BlockSpec((2, DM), full),
                  pl.BlockSpec((DM, DM), full)],
        out_specs=pl.BlockSpec((NB, DM), row),
        out_shape=jax.ShapeDtypeStruct((NV, DM), _f32),
    )(q, x, kv, ks, p['fc'], lnp, bd)
    return out


def _projkvq_body(x_ref, wk_ref, wv_ref, wq_ref, kv_ref, q_ref):
    x = x_ref[...]
    kv_ref[:, 0:DM] = jnp.dot(x, wk_ref[...], preferred_element_type=_f32)
    kv_ref[:, DM:2 * DM] = jnp.dot(x, wv_ref[...], preferred_element_type=_f32)
    q_ref[...] = jnp.dot(x, wq_ref[...], preferred_element_type=_f32)


def _proj_kv_q(x, wk, wv, wq):
    row = lambda i: (i, 0)
    full = lambda i: (0, 0)
    return pl.pallas_call(
        _projkvq_body,
        grid=(NV // NB,),
        in_specs=[pl.BlockSpec((NB, DM), row)] + [pl.BlockSpec((DM, DM), full)] * 3,
        out_specs=[pl.BlockSpec((NB, 2 * DM), row), pl.BlockSpec((NB, DM), row)],
        out_shape=[jax.ShapeDtypeStruct((NV, 2 * DM), _f32),
                   jax.ShapeDtypeStruct((NV, DM), _f32)],
    )(x, wk, wv, wq)


def _proj2_body(x_ref, wa_ref, wb_ref, oa_ref, ob_ref):
    x = x_ref[...]
    oa_ref[...] = jnp.dot(x, wa_ref[...], preferred_element_type=_f32)
    ob_ref[...] = jnp.dot(x, wb_ref[...], preferred_element_type=_f32)


def _proj2(x, wa, wb):
    row = lambda i: (i, 0)
    full = lambda i: (0, 0)
    return pl.pallas_call(
        _proj2_body,
        grid=(NE // EB,),
        in_specs=[pl.BlockSpec((EB, DM), row)] + [pl.BlockSpec((DM, DM), full)] * 2,
        out_specs=[pl.BlockSpec((EB, DM), row)] * 2,
        out_shape=[jax.ShapeDtypeStruct((NE, DM), _f32)] * 2,
    )(x, wa, wb)


def _edge_body(q_ref, kv_ref, e_ref, bd_ref, o_ref):
    q = q_ref[...]
    k = kv_ref[:, 0:DM]
    v = kv_ref[:, DM:2 * DM]
    s = q * k * e_ref[...]
    srep = jnp.dot(s, bd_ref[...], preferred_element_type=_f32) * 0.25
    attn = jnp.exp(jnp.clip(srep, -5.0, 5.0))
    o_ref[0] = attn * v
    o_ref[1] = attn


def _edge_math(qs, kvt, e, bd):
    row = lambda i: (i, 0)
    full = lambda i: (0, 0)
    return pl.pallas_call(
        _edge_body,
        grid=(NE // EB,),
        in_specs=[pl.BlockSpec((EB, DM), row), pl.BlockSpec((EB, 2 * DM), row),
                  pl.BlockSpec((EB, DM), row), pl.BlockSpec((DM, DM), full)],
        out_specs=pl.BlockSpec((2, EB, DM), lambda i: (0, i, 0)),
        out_shape=jax.ShapeDtypeStruct((2, NE, DM), _f32),
    )(qs, kvt, e, bd)


def _sc_gather(qtab, kvtab, qidx, kvidx):
    mesh = plsc.VectorSubcoreMesh(core_axis_name="c", subcore_axis_name="s")

    @functools.partial(
        pl.kernel,
        out_type=[jax.ShapeDtypeStruct((NE, DM), _f32),
                  jax.ShapeDtypeStruct((NE, 2 * DM), _f32)],
        mesh=mesh,
    )
    def gk(q_hbm, kv_hbm, qi_hbm, ki_hbm, oq_hbm, okv_hbm):
        def body(qi_v, ki_v, oq_v, okv_v):
            def inner(sem_q, sem_kv):
                cq = pltpu.async_copy(q_hbm.at[qi_v.at[0]], oq_v, sem_q)
                ckv = pltpu.async_copy(kv_hbm.at[ki_v.at[0]], okv_v, sem_kv)
                cq.wait()
                ckv.wait()

            pl.run_scoped(inner, pltpu.SemaphoreType.DMA,
                          pltpu.SemaphoreType.DMA)

        pltpu.emit_pipeline(
            body,
            grid=(NE // GW,),
            in_specs=[pl.BlockSpec((1, GW), lambda i: (0, i)),
                      pl.BlockSpec((1, GW), lambda i: (0, i))],
            out_specs=[pl.BlockSpec((GW, DM), lambda i: (i, 0)),
                       pl.BlockSpec((GW, 2 * DM), lambda i: (i, 0))],
            core_axis_name=("c", "s"),
            dimension_semantics=(pltpu.PARALLEL,),
        )(qi_hbm, ki_hbm, oq_hbm, okv_hbm)

    return gk(qtab, kvtab, qidx.reshape(1, NE), kvidx.reshape(1, NE))


def _sc_scatter(msg2, sidx):
    mesh = plsc.VectorSubcoreMesh(core_axis_name="c", subcore_axis_name="s")

    @functools.partial(
        pl.kernel,
        out_type=jax.ShapeDtypeStruct((2, NPAD, DM), _f32),
        mesh=mesh,
        scratch_types=[
            pltpu.VMEM((64, DM), _f32),
            pltpu.VMEM_SHARED((NPAD, DM), _f32),
        ],
    )
    def sk(m_hbm, si_hbm, o_hbm, z_v, acc_s):
        c = lax.axis_index("c")
        s = lax.axis_index("s")

        @pl.loop(0, 64)
        def _(r):
            for cc in range(DM // 16):
                z_v[r, pl.ds(cc * 16, 16)] = jnp.zeros((16,), _f32)

        @pl.loop(0, NPAD // NSUB // 64)
        def _(z):
            pltpu.sync_copy(z_v, acc_s.at[pl.ds(s * (NPAD // NSUB) + z * 64, 64)])

        plsc.subcore_barrier()

        def body(idx_p, row_p):
            pltpu.sync_copy(row_p.at[0], acc_s.at[idx_p.at[0]], add=True)

        for plane in range(2):
            @pl.when(c == plane)
            def _():
                pltpu.emit_pipeline(
                    body,
                    grid=(NE // SW,),
                    in_specs=[pl.BlockSpec((1, SW), lambda i: (0, i)),
                              pl.BlockSpec((1, SW, DM),
                                           lambda i, p=plane: (p, i, 0))],
                    out_specs=[],
                    core_axis_name=("s",),
                    dimension_semantics=(pltpu.PARALLEL,),
                )(si_hbm, m_hbm)

        plsc.subcore_barrier()
        r0 = s * (NPAD // NSUB)
        pltpu.sync_copy(acc_s.at[pl.ds(r0, NPAD // NSUB)],
                        o_hbm.at[c, pl.ds(r0, NPAD // NSUB)])

    return sk(msg2, sidx.reshape(1, NE))


def _post_body(m_ref, a_ref, xq_ref, fc_ref, lnp_ref,
               fc1_ref, fc2_ref, ln2_ref, o_ref):
    out = m_ref[0] / (a_ref[0] + 1e-8)
    y = jnp.dot(out, fc_ref[...], preferred_element_type=_f32) + xq_ref[...]
    y = _ln(y, lnp_ref[0:1, :], lnp_ref[1:2, :])
    h = jnp.maximum(jnp.dot(y, fc1_ref[...], preferred_element_type=_f32), 0.0)
    z = jnp.dot(h, fc2_ref[...], preferred_element_type=_f32) + y
    o_ref[...] = _ln(z, ln2_ref[0:1, :], ln2_ref[1:2, :])


def _post_ffn(acc, xq, pca, pffn):
    row = lambda i: (i, 0)
    full = lambda i: (0, 0)
    lnp = jnp.stack([pca['ln_g'], pca['ln_b']])
    ln2 = jnp.stack([pffn['ln_g'], pffn['ln_b']])
    return pl.pallas_call(
        _post_body,
        grid=(NV // NB,),
        in_specs=[pl.BlockSpec((1, NB, DM), lambda i: (0, i, 0)),
                  pl.BlockSpec((1, NB, DM), lambda i: (1, i, 0)),
                  pl.BlockSpec((NB, DM), row),
                  pl.BlockSpec((DM, DM), full),
                  pl.BlockSpec((2, DM), full),
                  pl.BlockSpec((DM, DFF), full),
                  pl.BlockSpec((DFF, DM), full),
                  pl.BlockSpec((2, DM), full)],
        out_specs=pl.BlockSpec((NB, DM), row),
        out_shape=jax.ShapeDtypeStruct((NV, DM), _f32),
    )(acc, acc, xq, pca['fc'], lnp, pffn['fc1'], pffn['fc2'], ln2)


def kernel(edge_indices, edge_features, var_features, con_features, params):
    src = edge_indices[0, 0]
    tgt = edge_indices[0, 1]
    ef = edge_features[0]
    xv = var_features[0]
    xc = con_features[0]
    p = params

    hid = np.arange(DM) // DK
    bd = jnp.asarray((hid[:, None] == hid[None, :]).astype(np.float32))

    var1 = _self_attn(xv, p['sa_var'], bd)
    con1 = _self_attn(xc, p['sa_con'], bd)

    kv_v2c, q_c2v = _proj_kv_q(var1, p['ca_v2c']['WK'], p['ca_v2c']['WV'],
                               p['ca_c2v']['WQ'])
    kv_c2v, q_v2c = _proj_kv_q(con1, p['ca_c2v']['WK'], p['ca_c2v']['WV'],
                               p['ca_v2c']['WQ'])
    e_v2c, e_c2v = _proj2(ef, p['ca_v2c']['WE'], p['ca_c2v']['WE'])

    qs_v2c, kvt_v2c = _sc_gather(q_v2c, kv_v2c, src, tgt)
    msg_v2c = _edge_math(qs_v2c, kvt_v2c, e_v2c, bd)
    acc_v2c = _sc_scatter(msg_v2c, src)

    qs_c2v, kvt_c2v = _sc_gather(q_c2v, kv_c2v, tgt, src)
    msg_c2v = _edge_math(qs_c2v, kvt_c2v, e_c2v, bd)
    acc_c2v = _sc_scatter(msg_c2v, tgt)

    con_out = _post_ffn(acc_v2c, con1, p['ca_v2c'], p['ffn_con'])
    var_out = _post_ffn(acc_c2v, var1, p['ca_c2v'], p['ffn_var'])

    return (var_out[None], con_out[None])
